# baseline (speedup 1.0000x reference)
import functools

import jax
import jax.numpy as jnp
from jax import lax
from jax.experimental import pallas as pl
from jax.experimental.pallas import tpu as pltpu
from jax.experimental.pallas import tpu_sc as plsc

NUM_CORES = 2
NUM_SUBCORES = 16
LANES = 16
NW = NUM_CORES * NUM_SUBCORES

BATCH = 16384
EMBED_DIM = 32
BPW = BATCH // NW
GROUPS = BPW // LANES


def _cf_body(uidx_hbm, iidx_hbm, uemb_hbm, iemb_hbm, ubias_hbm, ibias_hbm,
             out_hbm, uidx_v, iidx_v, urows_v, irows_v, ubias_v, ibias_v,
             out_v, sem_u, sem_i, sem_ub, sem_ib):
    wid = lax.axis_index("c") * NUM_SUBCORES + lax.axis_index("s")
    base = wid * BPW

    pltpu.sync_copy(uidx_hbm.at[pl.ds(base, BPW)], uidx_v)
    pltpu.sync_copy(iidx_hbm.at[pl.ds(base, BPW)], iidx_v)

    cu = pltpu.async_copy(uemb_hbm.at[uidx_v], urows_v, sem_u)
    ci = pltpu.async_copy(iemb_hbm.at[iidx_v], irows_v, sem_i)
    cub = pltpu.async_copy(ubias_hbm.at[uidx_v], ubias_v, sem_ub)
    cib = pltpu.async_copy(ibias_hbm.at[iidx_v], ibias_v, sem_ib)
    cu.wait()
    ci.wait()
    cub.wait()
    cib.wait()

    lanes = lax.iota(jnp.int32, LANES)

    def group_body(g, carry):
        acc = ubias_v[pl.ds(g * LANES, LANES)] + ibias_v[pl.ds(g * LANES, LANES)]
        rows = lanes + g * LANES
        for d in range(EMBED_DIM):
            col = jnp.full((LANES,), d, jnp.int32)
            u = plsc.load_gather(urows_v, [rows, col])
            v = plsc.load_gather(irows_v, [rows, col])
            acc = acc + u * v
        out_v[pl.ds(g * LANES, LANES)] = acc
        return carry

    lax.fori_loop(0, GROUPS, group_body, 0)

    pltpu.sync_copy(out_v, out_hbm.at[pl.ds(base, BPW)])


_cf_kernel = pl.kernel(
    _cf_body,
    out_type=jax.ShapeDtypeStruct((BATCH,), jnp.float32),
    mesh=plsc.VectorSubcoreMesh(core_axis_name="c", subcore_axis_name="s"),
    compiler_params=pltpu.CompilerParams(needs_layout_passes=False,
                                         use_tc_tiling_on_sc=False),
    scratch_types=[
        pltpu.VMEM((BPW,), jnp.int32),
        pltpu.VMEM((BPW,), jnp.int32),
        pltpu.VMEM((BPW, EMBED_DIM), jnp.float32),
        pltpu.VMEM((BPW, EMBED_DIM), jnp.float32),
        pltpu.VMEM((BPW,), jnp.float32),
        pltpu.VMEM((BPW,), jnp.float32),
        pltpu.VMEM((BPW,), jnp.float32),
        pltpu.SemaphoreType.DMA,
        pltpu.SemaphoreType.DMA,
        pltpu.SemaphoreType.DMA,
        pltpu.SemaphoreType.DMA,
    ],
)


@jax.jit
def kernel(user_indices, item_indices, user_emb_table, item_emb_table,
           user_bias_table, item_bias_table):
    return _cf_kernel(user_indices, item_indices, user_emb_table,
                      item_emb_table, user_bias_table.reshape(-1),
                      item_bias_table.reshape(-1))
